# Initial kernel scaffold; baseline (speedup 1.0000x reference)
import jax
import jax.numpy as jnp
from jax.experimental import pallas as pl


def kernel(features, emb_fm, lin_fm, bias, emb_dnn, W1, b1, W2, b2, W3, b3, Wf, bf):
    raise NotImplementedError("write your pallas kernel here")


# speedup vs baseline: 16.6242x; 16.6242x over previous
import functools

import jax
import jax.numpy as jnp
from jax import lax
from jax.experimental import pallas as pl
from jax.experimental.pallas import tpu as pltpu
from jax.experimental.pallas import tpu_sc as plsc

B = 4096
F = 26
V = 1000
D = 128
DNN_IN = F * D
H1, H2 = 1024, 512

NC = 2
NS = 16
NW = NC * NS
BPW = B // NW
IPW = BPW * F
CH = 4
CHI = CH * F
NCHUNK = BPW // CH

BB = 256
NBLK = B // BB


def _sc_gather(dnn_tab, fm_tab, lin_tab, idx, lin_idx):
    mesh = plsc.VectorSubcoreMesh(core_axis_name="c", subcore_axis_name="s",
                                  num_cores=NC, num_subcores=NS)

    @functools.partial(
        pl.kernel,
        mesh=mesh,
        compiler_params=pltpu.CompilerParams(needs_layout_passes=False),
        out_type=(
            jax.ShapeDtypeStruct((B * F, D), jnp.float32),
            jax.ShapeDtypeStruct((B * F, D), jnp.float32),
            jax.ShapeDtypeStruct((B,), jnp.float32),
        ),
        scratch_types=[
            pltpu.VMEM((IPW,), jnp.int32),
            pltpu.VMEM((CHI, D), jnp.float32),
            pltpu.VMEM((CHI, D), jnp.float32),
            pltpu.VMEM((F * V,), jnp.float32),
            pltpu.VMEM((F, BPW), jnp.int32),
            pltpu.VMEM((BPW,), jnp.float32),
            pltpu.SemaphoreType.DMA,
            pltpu.SemaphoreType.DMA,
        ],
    )
    def k(dnn_hbm, fm_hbm, lin_hbm, idx_hbm, lin_idx_hbm,
          dnn_out, fm_out, lin_out,
          idx_v, dnn_v, fm_v, lin_tab_v, lin_idx_v, lin_sum_v, sem1, sem2):
        wid = lax.axis_index("s") * NC + lax.axis_index("c")
        base = wid * IPW

        pltpu.sync_copy(idx_hbm.at[pl.ds(base, IPW)], idx_v)
        pltpu.sync_copy(lin_hbm, lin_tab_v)
        pltpu.sync_copy(lin_idx_hbm.at[:, pl.ds(wid * BPW, BPW)], lin_idx_v)

        for g in range(BPW // 16):
            acc = jnp.zeros((16,), jnp.float32)
            for f in range(F):
                iv = lin_idx_v[f, pl.ds(g * 16, 16)]
                acc = acc + plsc.load_gather(lin_tab_v, [iv])
            lin_sum_v[pl.ds(g * 16, 16)] = acc
        pltpu.sync_copy(lin_sum_v, lin_out.at[pl.ds(wid * BPW, BPW)])

        def chunk(j, _):
            off = j * CHI
            cp1 = pltpu.async_copy(dnn_hbm.at[idx_v.at[pl.ds(off, CHI)]],
                                   dnn_v, sem1)
            cp2 = pltpu.async_copy(fm_hbm.at[idx_v.at[pl.ds(off, CHI)]],
                                   fm_v, sem2)
            cp1.wait()
            cp2.wait()
            pltpu.sync_copy(dnn_v, dnn_out.at[pl.ds(base + off, CHI)])
            pltpu.sync_copy(fm_v, fm_out.at[pl.ds(base + off, CHI)])
            return 0

        lax.fori_loop(0, NCHUNK, chunk, 0)

    return k(dnn_tab, fm_tab, lin_tab, idx, lin_idx)


def _tc_body(dnn_x_ref, fm_x_ref, lin_ref, w1_ref, b1_ref, w2_ref, b2_ref,
             w3_ref, scal_ref, out_ref):
    x = dnn_x_ref[...]
    h = jnp.dot(x, w1_ref[...], preferred_element_type=jnp.float32)
    h = jnp.maximum(h + b1_ref[...], 0.0)
    h = jnp.dot(h, w2_ref[...], preferred_element_type=jnp.float32)
    h = jnp.maximum(h + b2_ref[...], 0.0)
    dnn_mat = jnp.dot(h, w3_ref[...], preferred_element_type=jnp.float32)
    dnn = jnp.sum(dnn_mat, axis=1)

    fm_x = fm_x_ref[...]
    s = jnp.zeros((BB, D), jnp.float32)
    q = jnp.zeros((BB, D), jnp.float32)
    for f in range(F):
        e = fm_x[:, f * D:(f + 1) * D]
        s = s + e
        q = q + e * e
    fm_term = 0.5 * jnp.sum(s * s - q, axis=1)

    bias0 = scal_ref[0]
    b3 = scal_ref[1]
    wf0 = scal_ref[2]
    wf1 = scal_ref[3]
    bf = scal_ref[4]
    fm_output = bias0 + lin_ref[...] + fm_term
    logit = wf0 * fm_output + wf1 * (dnn + b3) + bf
    out_ref[...] = 1.0 / (1.0 + jnp.exp(-logit))


def _tc_forward(dnn_x, fm_x, lin_sum, W1, b1, W2, b2, W3p, scal):
    return pl.pallas_call(
        _tc_body,
        grid=(NBLK,),
        in_specs=[
            pl.BlockSpec((BB, DNN_IN), lambda i: (i, 0)),
            pl.BlockSpec((BB, DNN_IN), lambda i: (i, 0)),
            pl.BlockSpec((BB,), lambda i: (i,)),
            pl.BlockSpec((DNN_IN, H1), lambda i: (0, 0)),
            pl.BlockSpec((1, H1), lambda i: (0, 0)),
            pl.BlockSpec((H1, H2), lambda i: (0, 0)),
            pl.BlockSpec((1, H2), lambda i: (0, 0)),
            pl.BlockSpec((H2, D), lambda i: (0, 0)),
            pl.BlockSpec(memory_space=pltpu.SMEM),
        ],
        out_specs=pl.BlockSpec((BB,), lambda i: (i,)),
        out_shape=jax.ShapeDtypeStruct((B,), jnp.float32),
    )(dnn_x, fm_x, lin_sum, W1, b1, W2, b2, W3p, scal)


def kernel(features, emb_fm, lin_fm, bias, emb_dnn, W1, b1, W2, b2, W3, b3,
           Wf, bf):
    feats = features.astype(jnp.int32)
    offs = (jnp.arange(F, dtype=jnp.int32) * V)[None, :]
    idx = (feats + offs).reshape(B * F)
    lin_idx = feats.T + (jnp.arange(F, dtype=jnp.int32) * V)[:, None]

    dnn_rows, fm_rows, lin_sum = _sc_gather(
        emb_dnn.reshape(F * V, D), emb_fm.reshape(F * V, D),
        lin_fm.reshape(F * V), idx, lin_idx)

    W3p = jnp.pad(W3, ((0, 0), (0, D - 1)))
    scal = jnp.concatenate([bias, b3, Wf[0], Wf[1], bf])
    out = _tc_forward(dnn_rows.reshape(B, DNN_IN), fm_rows.reshape(B, DNN_IN),
                      lin_sum, W1, b1.reshape(1, H1), W2, b2.reshape(1, H2),
                      W3p, scal)
    return out
